# Initial kernel scaffold; baseline (speedup 1.0000x reference)
import jax
import jax.numpy as jnp
from jax.experimental import pallas as pl


def kernel(x, token_table, pos_table):
    raise NotImplementedError("write your pallas kernel here")


# speedup vs baseline: 4.8121x; 4.8121x over previous
import functools

import jax
import jax.numpy as jnp
from jax import lax
from jax.experimental import pallas as pl
from jax.experimental.pallas import tpu as pltpu
from jax.experimental.pallas import tpu_sc as plsc


@functools.lru_cache(maxsize=None)
def _make_kernel(B, S, D, V):
    info = plsc.get_sparse_core_info()
    NC, NS, L = info.num_cores, info.num_subcores, info.num_lanes
    NW = NC * NS
    BPW = B // NW
    NCHUNK = 2
    H = S // NCHUNK
    assert B % NW == 0 and S % NCHUNK == 0 and H <= 128 and D % L == 0

    mesh = plsc.VectorSubcoreMesh(core_axis_name="c", subcore_axis_name="s")

    @functools.partial(
        pl.kernel,
        mesh=mesh,
        out_type=jax.ShapeDtypeStruct((B, NCHUNK, H, D), jnp.float32),
        scratch_types=[
            pltpu.VMEM((BPW, NCHUNK, H), jnp.int32),
            pltpu.VMEM((NCHUNK, H, D), jnp.float32),
            pltpu.VMEM((NCHUNK, H, D), jnp.float32),
            pltpu.SemaphoreType.DMA,
        ],
    )
    def k(x_hbm, tok_hbm, pos_hbm, out_hbm, idx_v, pos_v, rows_v, sem):
        wid = lax.axis_index("s") * NC + lax.axis_index("c")
        b0 = wid * BPW
        pltpu.sync_copy(pos_hbm, pos_v)
        pltpu.sync_copy(x_hbm.at[pl.ds(b0, BPW)], idx_v)

        def row_body(i, carry):
            for j in range(NCHUNK):
                pltpu.async_copy(tok_hbm.at[idx_v.at[i, j]], rows_v.at[j],
                                 sem).wait()

            def add_body(r, c):
                for j in range(NCHUNK):
                    for v in range(D // L):
                        sl = pl.ds(v * L, L)
                        plsc.addupdate(rows_v.at[j, r, sl], pos_v[j, r, sl])
                return c

            lax.fori_loop(0, H, add_body, 0)
            pltpu.sync_copy(rows_v, out_hbm.at[b0 + i])
            return carry

        lax.fori_loop(0, BPW, row_body, 0)

    return k


def kernel(x, token_table, pos_table):
    B, S = x.shape
    V, D = token_table.shape
    x2 = x.astype(jnp.int32).reshape(B, 2, S // 2)
    pos2 = pos_table.reshape(2, S // 2, D)
    out = _make_kernel(B, S, D, V)(x2, token_table, pos2)
    return out.reshape(B, S, D)


# speedup vs baseline: 6.8840x; 1.4305x over previous
import functools

import jax
import jax.numpy as jnp
from jax import lax
from jax.experimental import pallas as pl
from jax.experimental.pallas import tpu as pltpu
from jax.experimental.pallas import tpu_sc as plsc


@functools.lru_cache(maxsize=None)
def _make_kernel(B, S, D, V):
    info = plsc.get_sparse_core_info()
    NC, NS, L = info.num_cores, info.num_subcores, info.num_lanes
    NW = NC * NS
    BPW = B // NW
    H = S // 2
    NCH = 2 * BPW
    NB = 4
    assert B % NW == 0 and S % 2 == 0 and H % 2 == 0 and H <= 128
    assert D % L == 0 and NCH % NB == 0 and NCH >= NB

    mesh = plsc.VectorSubcoreMesh(core_axis_name="c", subcore_axis_name="s")

    @functools.partial(
        pl.kernel,
        mesh=mesh,
        out_type=jax.ShapeDtypeStruct((B, 2, H, D), jnp.float32),
        scratch_types=[
            pltpu.VMEM((BPW, 2, H), jnp.int32),
            pltpu.VMEM((2, H, D), jnp.float32),
            pltpu.VMEM((NB, H, D), jnp.float32),
        ] + [pltpu.SemaphoreType.DMA] * (2 * NB),
    )
    def k(x_hbm, tok_hbm, pos_hbm, out_hbm, idx_v, pos_v, rows_v, *sems):
        sin, sout = sems[:NB], sems[NB:]
        wid = lax.axis_index("s") * NC + lax.axis_index("c")
        b0 = wid * BPW
        pltpu.sync_copy(pos_hbm, pos_v)
        pltpu.sync_copy(x_hbm.at[pl.ds(b0, BPW)], idx_v)

        def gather_start(row, j, buf):
            pltpu.async_copy(tok_hbm.at[idx_v.at[row, j]], rows_v.at[buf],
                             sin[buf])

        def gather_wait(row, j, buf):
            pltpu.make_async_copy(tok_hbm.at[idx_v.at[row, j]],
                                  rows_v.at[buf], sin[buf]).wait()

        def out_start(row, j, buf):
            pltpu.async_copy(rows_v.at[buf], out_hbm.at[b0 + row, j],
                             sout[buf])

        def out_wait(buf):
            pltpu.make_async_copy(rows_v.at[buf], out_hbm.at[b0, 0],
                                  sout[buf]).wait()

        def add_pos(buf, j):
            def body(r2, carry):
                for dr in range(2):
                    r = 2 * r2 + dr
                    for v in range(D // L):
                        sl = pl.ds(v * L, L)
                        plsc.addupdate(rows_v.at[buf, r, sl], pos_v[j, r, sl])
                return carry

            lax.fori_loop(0, H // 2, body, 0)

        gather_start(0, 0, 0)
        gather_start(0, 1, 1)

        def outer(t, carry):
            for b in range(NB):
                j = b % 2
                nb2 = (b + 2) % NB

                @pl.when(NB * t + b >= 2)
                def _():
                    out_wait(nb2)

                @pl.when(NB * t + b + 2 < NCH)
                def _():
                    gather_start(2 * t + (b + 2) // 2, j, nb2)

                gather_wait(2 * t + b // 2, j, b)
                add_pos(b, j)
                out_start(2 * t + b // 2, j, b)
            return carry

        lax.fori_loop(0, NCH // NB, outer, 0)
        out_wait((NCH - 2) % NB)
        out_wait((NCH - 1) % NB)

    return k


def kernel(x, token_table, pos_table):
    B, S = x.shape
    V, D = token_table.shape
    x2 = x.astype(jnp.int32).reshape(B, 2, S // 2)
    pos2 = pos_table.reshape(2, S // 2, D)
    out = _make_kernel(B, S, D, V)(x2, token_table, pos2)
    return out.reshape(B, S, D)


# speedup vs baseline: 7.4900x; 1.0880x over previous
import functools

import jax
import jax.numpy as jnp
from jax import lax
from jax.experimental import pallas as pl
from jax.experimental.pallas import tpu as pltpu
from jax.experimental.pallas import tpu_sc as plsc

_C = 128


@functools.lru_cache(maxsize=None)
def _make_kernel(B, S, D, V):
    info = plsc.get_sparse_core_info()
    NC, NS, L = info.num_cores, info.num_subcores, info.num_lanes
    NW = NC * NS
    T = B * S
    TPW = T // NW
    NCH = TPW // _C
    NB = 4
    UNROLL = 8
    assert T % (NW * _C) == 0 and TPW % S == 0 and D % L == 0
    assert S % UNROLL == 0 and _C % UNROLL == 0 and NCH % NB == 0
    assert NCH >= NB

    mesh = plsc.VectorSubcoreMesh(core_axis_name="c", subcore_axis_name="s")

    @functools.partial(
        pl.kernel,
        mesh=mesh,
        out_type=jax.ShapeDtypeStruct((T, D), jnp.float32),
        scratch_types=[
            pltpu.VMEM((NCH, _C), jnp.int32),
            pltpu.VMEM((S, D), jnp.float32),
            pltpu.VMEM((NB, _C, D), jnp.float32)
        ] + [pltpu.SemaphoreType.DMA] * (2 * NB),
    )
    def k(x_hbm, tok_hbm, pos_hbm, out_hbm, idx_v, pos_v, rows_v, *sems):
        sin, sout = sems[:NB], sems[NB:]
        wid = lax.axis_index("s") * NC + lax.axis_index("c")
        f0 = wid * TPW
        pltpu.sync_copy(pos_hbm, pos_v)
        pltpu.sync_copy(x_hbm.at[pl.ds(wid * NCH, NCH)], idx_v)

        def gather_start(c, buf):
            pltpu.async_copy(tok_hbm.at[idx_v.at[c]], rows_v.at[buf],
                             sin[buf])

        def gather_wait(c, buf):
            pltpu.make_async_copy(tok_hbm.at[idx_v.at[c]], rows_v.at[buf],
                                  sin[buf]).wait()

        def out_start(c, buf):
            pltpu.async_copy(rows_v.at[buf],
                             out_hbm.at[pl.ds(f0 + c * _C, _C)], sout[buf])

        def out_wait(buf):
            pltpu.make_async_copy(rows_v.at[buf], out_hbm.at[pl.ds(0, _C)],
                                  sout[buf]).wait()

        def add_pos(c, buf):
            s0 = lax.rem(c * _C, S)
            n1 = lax.min(S - s0, _C)

            def run(tok0, row0, nit):
                def body(i, carry):
                    for u in range(UNROLL):
                        tok = tok0 + i * UNROLL + u
                        row = row0 + i * UNROLL + u
                        for v in range(D // L):
                            sl = pl.ds(v * L, L)
                            plsc.addupdate(rows_v.at[buf, tok, sl],
                                           pos_v[row, sl])
                    return carry

                lax.fori_loop(0, nit, body, 0)

            run(0, s0, n1 // UNROLL)
            run(n1, 0, (_C - n1) // UNROLL)

        gather_start(0, 0)
        gather_start(1, 1)

        def outer(t, carry):
            for b in range(NB):
                c = NB * t + b
                nb2 = (b + 2) % NB

                @pl.when(c >= 2)
                def _():
                    out_wait(nb2)

                @pl.when(c + 2 < NCH)
                def _():
                    gather_start(c + 2, nb2)

                gather_wait(c, b)
                add_pos(c, b)
                out_start(c, b)
            return carry

        lax.fori_loop(0, NCH // NB, outer, 0)
        out_wait((NCH - 2) % NB)
        out_wait((NCH - 1) % NB)

    return k


def kernel(x, token_table, pos_table):
    B, S = x.shape
    V, D = token_table.shape
    x2 = x.astype(jnp.int32).reshape(B * S // _C, _C)
    out = _make_kernel(B, S, D, V)(x2, token_table, pos_table)
    return out.reshape(B, S, D)


# speedup vs baseline: 15.7137x; 2.0980x over previous
import functools

import jax
import jax.numpy as jnp
from jax import lax
from jax.experimental import pallas as pl
from jax.experimental.pallas import tpu as pltpu
from jax.experimental.pallas import tpu_sc as plsc


@functools.lru_cache(maxsize=None)
def _make_kernel(B, S, D, V):
    info = plsc.get_sparse_core_info()
    NC, NS, L = info.num_cores, info.num_subcores, info.num_lanes
    NW = NC * NS
    BPW = B // NW
    H = S // 2
    NB = 2
    assert B % NW == 0 and S % 8 == 0 and H <= 128 and D % L == 0
    assert BPW % NB == 0

    mesh = plsc.VectorSubcoreMesh(core_axis_name="c", subcore_axis_name="s")

    @functools.partial(
        pl.kernel,
        mesh=mesh,
        out_type=jax.ShapeDtypeStruct((B * S, D), jnp.float32),
        scratch_types=[
            pltpu.VMEM((BPW, 2, H), jnp.int32),
            pltpu.VMEM((S, D), jnp.float32),
            pltpu.VMEM((NB, S, D), jnp.float32),
        ] + [pltpu.SemaphoreType.DMA] * (2 * NB),
    )
    def k(x_hbm, tok_hbm, pos_hbm, out_hbm, idx_v, pos_v, rows_v, *sems):
        sin, sout = sems[:NB], sems[NB:]
        wid = lax.axis_index("s") * NC + lax.axis_index("c")
        b0 = wid * BPW
        pltpu.sync_copy(pos_hbm, pos_v)
        pltpu.sync_copy(x_hbm.at[pl.ds(b0, BPW)], idx_v)

        def gather_copies(i, buf):
            return [
                pltpu.make_async_copy(tok_hbm.at[idx_v.at[i, j]],
                                      rows_v.at[buf, pl.ds(j * H, H)],
                                      sin[buf])
                for j in range(2)
            ]

        def gather_start(i, buf):
            for cp in gather_copies(i, buf):
                cp.start()

        def gather_wait(i, buf):
            for cp in gather_copies(i, buf):
                cp.wait()

        def out_start(i, buf):
            pltpu.async_copy(rows_v.at[buf],
                             out_hbm.at[pl.ds((b0 + i) * S, S)], sout[buf])

        def out_wait(buf):
            pltpu.make_async_copy(rows_v.at[buf], out_hbm.at[pl.ds(0, S)],
                                  sout[buf]).wait()

        def add_pos(buf):
            def body(r2, carry):
                for dr in range(2):
                    r = 2 * r2 + dr
                    for v in range(D // L):
                        sl = pl.ds(v * L, L)
                        plsc.addupdate(rows_v.at[buf, r, sl], pos_v[r, sl])
                return carry

            lax.fori_loop(0, S // 2, body, 0)

        gather_start(0, 0)

        def outer(t, carry):
            for b in range(NB):
                i = NB * t + b
                nb1 = (b + 1) % NB

                @pl.when(i >= 1)
                def _():
                    out_wait(nb1)

                @pl.when(i + 1 < BPW)
                def _():
                    gather_start(i + 1, nb1)

                gather_wait(i, b)
                add_pos(b)
                out_start(i, b)
            return carry

        lax.fori_loop(0, BPW // NB, outer, 0)
        out_wait((BPW - 1) % NB)

    return k


def kernel(x, token_table, pos_table):
    B, S = x.shape
    V, D = token_table.shape
    x2 = x.astype(jnp.int32).reshape(B, 2, S // 2)
    out = _make_kernel(B, S, D, V)(x2, token_table, pos_table)
    return out.reshape(B, S, D)


# speedup vs baseline: 18.0435x; 1.1483x over previous
import functools

import jax
import jax.numpy as jnp
from jax import lax
from jax.experimental import pallas as pl
from jax.experimental.pallas import tpu as pltpu
from jax.experimental.pallas import tpu_sc as plsc

_SEG = (96, 104)


@functools.lru_cache(maxsize=None)
def _make_kernel(B, S, D, V):
    info = plsc.get_sparse_core_info()
    NC, NS, L = info.num_cores, info.num_subcores, info.num_lanes
    NW = NC * NS
    BPW = B // NW
    NB = 2
    segs = []
    o = 0
    for n in _SEG:
        segs.append((o, n))
        o += n
    assert o == S and all(n % 8 == 0 and s % 8 == 0 and n <= 128
                          for s, n in segs)
    assert B % NW == 0 and BPW % NB == 0 and D % L == 0

    mesh = plsc.VectorSubcoreMesh(core_axis_name="c", subcore_axis_name="s")

    @functools.partial(
        pl.kernel,
        mesh=mesh,
        out_type=jax.ShapeDtypeStruct((B * S, D), jnp.float32),
        scratch_types=[
            pltpu.VMEM((BPW, 2, S // 2), jnp.int32),
            pltpu.VMEM((S, D), jnp.float32),
            pltpu.VMEM((NB, S, D), jnp.float32),
        ] + [pltpu.SemaphoreType.DMA] * (2 * NB),
    )
    def k(x_hbm, tok_hbm, pos_hbm, out_hbm, idx_v, pos_v, rows_v, *sems):
        sin, sout = sems[:NB], sems[NB:]
        wid = lax.axis_index("s") * NC + lax.axis_index("c")
        b0 = wid * BPW
        pltpu.sync_copy(pos_hbm, pos_v)
        pltpu.sync_copy(x_hbm.at[pl.ds(b0, BPW)], idx_v)

        H = S // 2

        def gather_copy(i, buf, j):
            return pltpu.make_async_copy(
                tok_hbm.at[idx_v.at[i, j]],
                rows_v.at[buf, pl.ds(j * H, H)], sin[buf])

        def gather_start(i, buf):
            for j in range(2):
                gather_copy(i, buf, j).start()

        def gather_wait(i, buf):
            for j in range(2):
                gather_copy(i, buf, j).wait()

        def out_start(i, buf, seg):
            s0, n = segs[seg]
            pltpu.async_copy(rows_v.at[buf, pl.ds(s0, n)],
                             out_hbm.at[pl.ds((b0 + i) * S + s0, n)],
                             sout[buf])

        def out_wait(buf):
            for s0, n in segs:
                pltpu.make_async_copy(rows_v.at[buf, pl.ds(s0, n)],
                                      out_hbm.at[pl.ds(s0, n)],
                                      sout[buf]).wait()

        def add_pos(buf, seg):
            s0, n = segs[seg]

            def body(r2, carry):
                for dr in range(2):
                    r = s0 + 2 * r2 + dr
                    for v in range(D // L):
                        sl = pl.ds(v * L, L)
                        plsc.addupdate(rows_v.at[buf, r, sl], pos_v[r, sl])
                return carry

            lax.fori_loop(0, n // 2, body, 0)

        gather_start(0, 0)

        def outer(t, carry):
            for b in range(NB):
                i = NB * t + b
                nb1 = (b + 1) % NB

                @pl.when(i >= 1)
                def _():
                    out_wait(nb1)

                @pl.when(i + 1 < BPW)
                def _():
                    gather_start(i + 1, nb1)

                gather_wait(i, b)
                for seg in range(len(segs)):
                    add_pos(b, seg)
                    out_start(i, b, seg)
            return carry

        lax.fori_loop(0, BPW // NB, outer, 0)
        out_wait((BPW - 1) % NB)

    return k


def kernel(x, token_table, pos_table):
    B, S = x.shape
    V, D = token_table.shape
    x2 = x.astype(jnp.int32).reshape(B, 2, S // 2)
    out = _make_kernel(B, S, D, V)(x2, token_table, pos_table)
    return out.reshape(B, S, D)
